# baseline (speedup 1.0000x reference)
import functools

import jax
import jax.numpy as jnp
from jax import lax
from jax.experimental import pallas as pl
from jax.experimental.pallas import tpu as pltpu
from jax.experimental.pallas import tpu_sc as plsc

B = 16384
F = 26
V = 100000
D = 32
BF = B * F

NC, NS = 2, 16
NW = NC * NS
BPW = BF // NW
GSZ = 128
CHUNK = 1024
NG = CHUNK // GSZ
NCHUNK = BPW // CHUNK
IDX_ROWS = BF // GSZ


def _make_kernel():
    mesh = plsc.VectorSubcoreMesh(core_axis_name="c", subcore_axis_name="s")

    @functools.partial(
        pl.kernel,
        mesh=mesh,
        out_type=jax.ShapeDtypeStruct((BF, D), jnp.float32),
        compiler_params=pltpu.CompilerParams(use_tc_tiling_on_sc=False),
        scratch_types=[
            pltpu.VMEM((CHUNK,), jnp.int32),
            pltpu.VMEM((CHUNK, D), jnp.float32),
            pltpu.SemaphoreType.DMA,
        ],
    )
    def gather_kernel(idx_hbm, table_hbm, out_hbm, idx_v, rows_v, sem):
        wid = lax.axis_index("s") * NC + lax.axis_index("c")
        base = wid * BPW

        def chunk_body(c, carry):
            pltpu.sync_copy(idx_hbm.at[pl.ds(base + c * CHUNK, CHUNK)], idx_v)
            pltpu.async_copy(table_hbm.at[idx_v], rows_v, sem).wait()
            pltpu.sync_copy(
                rows_v, out_hbm.at[pl.ds(base + c * CHUNK, CHUNK)]
            )
            return carry

        lax.fori_loop(0, NCHUNK, chunk_body, 0)

    return gather_kernel


_gather = _make_kernel()


def kernel(indices, tables):
    flat_tables = tables.reshape(F * V, D)
    offsets = (jnp.arange(F, dtype=indices.dtype) * V)[None, :]
    flat_idx = (indices + offsets).reshape(BF)
    out = _gather(flat_idx, flat_tables)
    return out.reshape(B, F * D)


# speedup vs baseline: 3.4780x; 3.4780x over previous
import functools

import jax
import jax.numpy as jnp
from jax import lax
from jax.experimental import pallas as pl
from jax.experimental.pallas import tpu as pltpu
from jax.experimental.pallas import tpu_sc as plsc

B = 16384
F = 26
V = 100000
D = 32

NC, NS = 2, 16
NW = NC * NS
BH = B // 2


def _make_kernel():
    mesh = plsc.VectorSubcoreMesh(core_axis_name="c", subcore_axis_name="s")

    @functools.partial(
        pl.kernel,
        mesh=mesh,
        out_type=jax.ShapeDtypeStruct((F * D, B), jnp.float32),
        compiler_params=pltpu.CompilerParams(needs_layout_passes=False),
        scratch_types=[
            pltpu.VMEM((V,), jnp.float32),
            pltpu.VMEM((BH,), jnp.int32),
            pltpu.VMEM((BH,), jnp.float32),
        ],
    )
    def gather_kernel(idx_hbm, tab_hbm, out_hbm, row_v, idx_v, out_v):
        d = lax.axis_index("s") * NC + lax.axis_index("c")

        def field_body(f, carry):
            pltpu.sync_copy(tab_hbm.at[f, d], row_v)

            def half_body(h, carry2):
                pltpu.sync_copy(idx_hbm.at[f, pl.ds(h * BH, BH)], idx_v)

                def vec_body(i, carry3):
                    iv = idx_v[pl.ds(i * 16, 16)]
                    out_v[pl.ds(i * 16, 16)] = plsc.load_gather(row_v, [iv])
                    return carry3

                lax.fori_loop(0, BH // 16, vec_body, 0)
                pltpu.sync_copy(
                    out_v, out_hbm.at[f * D + d, pl.ds(h * BH, BH)]
                )
                return carry2

            lax.fori_loop(0, 2, half_body, 0)
            return carry

        lax.fori_loop(0, F, field_body, 0)

    return gather_kernel


_gather = _make_kernel()


def kernel(indices, tables):
    idx_t = indices.T
    tab_t = tables.transpose(0, 2, 1)
    out_t = _gather(idx_t, tab_t)
    return out_t.T


# speedup vs baseline: 5.0984x; 1.4659x over previous
import functools

import jax
import jax.numpy as jnp
from jax import lax
from jax.experimental import pallas as pl
from jax.experimental.pallas import tpu as pltpu
from jax.experimental.pallas import tpu_sc as plsc

B = 16384
F = 26
V = 100000
D = 32

NC, NS = 2, 16
NW = NC * NS
BH = B // 2


def _make_kernel():
    mesh = plsc.VectorSubcoreMesh(core_axis_name="c", subcore_axis_name="s")

    @functools.partial(
        pl.kernel,
        mesh=mesh,
        out_type=jax.ShapeDtypeStruct((F * D, B), jnp.float32),
        compiler_params=pltpu.CompilerParams(needs_layout_passes=False),
        scratch_types=[
            pltpu.VMEM((V,), jnp.float32),
            pltpu.VMEM((BH,), jnp.int32),
            pltpu.VMEM((BH,), jnp.float32),
        ],
    )
    def gather_kernel(idx_hbm, tab_hbm, out_hbm, row_v, idx_v, out_v):
        d = lax.axis_index("s") * NC + lax.axis_index("c")

        def field_body(f, carry):
            pltpu.sync_copy(tab_hbm.at[f, d], row_v)

            def half_body(h, carry2):
                pltpu.sync_copy(idx_hbm.at[f, pl.ds(h * BH, BH)], idx_v)

                @plsc.parallel_loop(0, BH, 16, unroll=8)
                def vec_body(i):
                    iv = idx_v[pl.ds(i, 16)]
                    out_v[pl.ds(i, 16)] = plsc.load_gather(row_v, [iv])
                pltpu.sync_copy(
                    out_v, out_hbm.at[f * D + d, pl.ds(h * BH, BH)]
                )
                return carry2

            lax.fori_loop(0, 2, half_body, 0)
            return carry

        lax.fori_loop(0, F, field_body, 0)

    return gather_kernel


_gather = _make_kernel()


def kernel(indices, tables):
    idx_t = indices.T
    tab_t = tables.transpose(0, 2, 1)
    out_t = _gather(idx_t, tab_t)
    return out_t.T


# speedup vs baseline: 5.6739x; 1.1129x over previous
import functools

import jax
import jax.numpy as jnp
from jax import lax
from jax.experimental import pallas as pl
from jax.experimental.pallas import tpu as pltpu
from jax.experimental.pallas import tpu_sc as plsc

B = 16384
F = 26
V = 100000
D = 32
BH = B // 2

NC, NS = 2, 16
NW = NC * NS


def _make_kernel():
    mesh = plsc.VectorSubcoreMesh(core_axis_name="c", subcore_axis_name="s")

    @functools.partial(
        pl.kernel,
        mesh=mesh,
        out_type=jax.ShapeDtypeStruct((F * D, B), jnp.float32),
        compiler_params=pltpu.CompilerParams(needs_layout_passes=False),
        scratch_types=[
            pltpu.VMEM((V,), jnp.float32),
            pltpu.VMEM((B,), jnp.int32),
            pltpu.VMEM((BH,), jnp.float32),
            pltpu.SemaphoreType.DMA,
            pltpu.SemaphoreType.DMA,
        ],
    )
    def gather_kernel(
        idx_hbm, tab_hbm, out_hbm, row_v, idx_v, out_v, sem_row, sem_idx
    ):
        d = lax.axis_index("s") * NC + lax.axis_index("c")

        pltpu.async_copy(tab_hbm.at[0, d], row_v, sem_row)
        pltpu.async_copy(idx_hbm.at[0], idx_v, sem_idx)

        def field_body(f, carry):
            pltpu.make_async_copy(tab_hbm.at[f, d], row_v, sem_row).wait()
            pltpu.make_async_copy(idx_hbm.at[f], idx_v, sem_idx).wait()

            for h in (0, 1):
                base = h * BH

                @plsc.parallel_loop(0, BH, 16, unroll=8)
                def vec_body(i):
                    iv = idx_v[pl.ds(base + i, 16)]
                    out_v[pl.ds(i, 16)] = plsc.load_gather(row_v, [iv])

                if h == 1:
                    @pl.when(f + 1 < F)
                    def _():
                        pltpu.async_copy(tab_hbm.at[f + 1, d], row_v, sem_row)
                        pltpu.async_copy(idx_hbm.at[f + 1], idx_v, sem_idx)
                pltpu.sync_copy(
                    out_v, out_hbm.at[f * D + d, pl.ds(base, BH)]
                )
            return carry

        lax.fori_loop(0, F, field_body, 0)

    return gather_kernel


_gather = _make_kernel()


def kernel(indices, tables):
    idx_t = indices.T
    tab_t = tables.transpose(0, 2, 1)
    out_t = _gather(idx_t, tab_t)
    return out_t.T


# speedup vs baseline: 5.6878x; 1.0025x over previous
import functools

import jax
import jax.numpy as jnp
from jax import lax
from jax.experimental import pallas as pl
from jax.experimental.pallas import tpu as pltpu
from jax.experimental.pallas import tpu_sc as plsc

B = 16384
F = 26
V = 100000
D = 32
BH = B // 2

NC, NS = 2, 16
NW = NC * NS


def _make_kernel():
    mesh = plsc.VectorSubcoreMesh(core_axis_name="c", subcore_axis_name="s")

    @functools.partial(
        pl.kernel,
        mesh=mesh,
        out_type=jax.ShapeDtypeStruct((F * D, B), jnp.float32),
        compiler_params=pltpu.CompilerParams(needs_layout_passes=False),
        scratch_types=[
            pltpu.VMEM((V,), jnp.float32),
            pltpu.VMEM((B,), jnp.int32),
            pltpu.VMEM((BH,), jnp.float32),
            pltpu.SemaphoreType.DMA,
            pltpu.SemaphoreType.DMA,
        ],
    )
    def gather_kernel(
        idx_hbm, tab_hbm, out_hbm, row_v, idx_v, out_v, sem_row, sem_idx
    ):
        d = lax.axis_index("s") * NC + lax.axis_index("c")

        pltpu.async_copy(tab_hbm.at[0, d], row_v, sem_row)
        pltpu.async_copy(idx_hbm.at[0], idx_v, sem_idx)

        def field_body(f, carry):
            pltpu.make_async_copy(tab_hbm.at[f, d], row_v, sem_row).wait()
            pltpu.make_async_copy(idx_hbm.at[f], idx_v, sem_idx).wait()

            for h in (0, 1):
                base = h * BH

                @plsc.parallel_loop(0, BH, 16, unroll=16)
                def vec_body(i):
                    iv = idx_v[pl.ds(base + i, 16)]
                    out_v[pl.ds(i, 16)] = plsc.load_gather(row_v, [iv])

                if h == 1:
                    @pl.when(f + 1 < F)
                    def _():
                        pltpu.async_copy(tab_hbm.at[f + 1, d], row_v, sem_row)
                        pltpu.async_copy(idx_hbm.at[f + 1], idx_v, sem_idx)
                pltpu.sync_copy(
                    out_v, out_hbm.at[f * D + d, pl.ds(base, BH)]
                )
            return carry

        lax.fori_loop(0, F, field_body, 0)

    return gather_kernel


_gather = _make_kernel()


def kernel(indices, tables):
    idx_t = indices.T
    tab_t = tables.transpose(0, 2, 1)
    out_t = _gather(idx_t, tab_t)
    return out_t.T
